# baseline (speedup 1.0000x reference)
import jax
import jax.numpy as jnp
from jax import lax
from jax.experimental import pallas as pl
from jax.experimental.pallas import tpu as pltpu
from jax.experimental.pallas import tpu_sc as plsc

_BATCH = 16384
_XDIM = 128
_EDIM = 10
_ODIM = _XDIM + _EDIM
_MAXP = 36
_PAD = 16
_NC, _NS = 2, 16
_NW = _NC * _NS
_BPW = _BATCH // _NW
_ICHUNK = 128
_NPC = _BPW // _ICHUNK
_XBLK = 16384


def _sc_gather_body(idx_hbm, tbl_hbm, out_hbm, idx_v, piece_v, gsem):
    wid = lax.axis_index("s") * _NC + lax.axis_index("c")

    pltpu.sync_copy(idx_hbm.at[pl.ds(wid * _NPC, _NPC)], idx_v)

    gathers = [
        pltpu.async_copy(tbl_hbm.at[idx_v.at[pc]], piece_v.at[pc], gsem)
        for pc in range(_NPC)
    ]
    for pc in range(_NPC):
        gathers[pc].wait()
        pltpu.sync_copy(
            piece_v.at[pc],
            out_hbm.at[pl.ds(wid * _BPW + pc * _ICHUNK, _ICHUNK)])


def _concat_body(x_ref, pe_ref, o_ref):
    o_ref[...] = jnp.concatenate(
        [x_ref[...], pe_ref[:, :_EDIM]], axis=1)


def kernel(x, position, emb):
    idx = position.astype(jnp.int32).reshape(_NW * _NPC, _ICHUNK)
    tbl = jnp.pad(emb, ((0, 0), (0, _PAD - _EDIM)))

    mesh = plsc.VectorSubcoreMesh(core_axis_name="c", subcore_axis_name="s")
    pe16 = pl.kernel(
        _sc_gather_body,
        out_type=jax.ShapeDtypeStruct((_BATCH, _PAD), jnp.float32),
        mesh=mesh,
        scratch_types=[
            pltpu.VMEM((_NPC, _ICHUNK), jnp.int32),
            pltpu.VMEM((_NPC, _ICHUNK, _PAD), jnp.float32),
            pltpu.SemaphoreType.DMA,
        ],
        compiler_params=pltpu.CompilerParams(use_tc_tiling_on_sc=False),
    )(idx, tbl)

    return pl.pallas_call(
        _concat_body,
        grid=(_BATCH // _XBLK,),
        in_specs=[
            pl.BlockSpec((_XBLK, _XDIM), lambda i: (i, 0)),
            pl.BlockSpec((_XBLK, _PAD), lambda i: (i, 0)),
        ],
        out_specs=pl.BlockSpec((_XBLK, _ODIM), lambda i: (i, 0)),
        out_shape=jax.ShapeDtypeStruct((_BATCH, _ODIM), jnp.float32),
    )(x, pe16)


# speedup vs baseline: 1.0200x; 1.0200x over previous
import jax
import jax.numpy as jnp
from jax import lax
from jax.experimental import pallas as pl
from jax.experimental.pallas import tpu as pltpu
from jax.experimental.pallas import tpu_sc as plsc

_BATCH = 16384
_XDIM = 128
_EDIM = 10
_ODIM = _XDIM + _EDIM
_MAXP = 36
_PAD = 16
_NC, _NS = 2, 16
_NW = _NC * _NS
_BPW = _BATCH // _NW
_ICHUNK = 128
_NPC = _BPW // _ICHUNK
_XBLK = 8192


def _sc_gather_body(idx_hbm, tbl_hbm, out_hbm, idx_v, piece_v, gsem):
    wid = lax.axis_index("s") * _NC + lax.axis_index("c")

    pltpu.sync_copy(idx_hbm.at[pl.ds(wid * _NPC, _NPC)], idx_v)

    gathers = [
        pltpu.async_copy(tbl_hbm.at[idx_v.at[pc]], piece_v.at[pc], gsem)
        for pc in range(_NPC)
    ]
    for pc in range(_NPC):
        gathers[pc].wait()
        pltpu.sync_copy(
            piece_v.at[pc],
            out_hbm.at[pl.ds(wid * _BPW + pc * _ICHUNK, _ICHUNK)])


def _concat_body(x_ref, pe_ref, o_ref):
    o_ref[...] = jnp.concatenate(
        [x_ref[...], pe_ref[:, :_EDIM]], axis=1)


def kernel(x, position, emb):
    idx = position.astype(jnp.int32).reshape(_NW * _NPC, _ICHUNK)
    tbl = jnp.pad(emb, ((0, 0), (0, _PAD - _EDIM)))

    mesh = plsc.VectorSubcoreMesh(core_axis_name="c", subcore_axis_name="s")
    pe16 = pl.kernel(
        _sc_gather_body,
        out_type=jax.ShapeDtypeStruct((_BATCH, _PAD), jnp.float32),
        mesh=mesh,
        scratch_types=[
            pltpu.VMEM((_NPC, _ICHUNK), jnp.int32),
            pltpu.VMEM((_NPC, _ICHUNK, _PAD), jnp.float32),
            pltpu.SemaphoreType.DMA,
        ],
        compiler_params=pltpu.CompilerParams(use_tc_tiling_on_sc=False),
    )(idx, tbl)

    return pl.pallas_call(
        _concat_body,
        grid=(_BATCH // _XBLK,),
        in_specs=[
            pl.BlockSpec((_XBLK, _XDIM), lambda i: (i, 0)),
            pl.BlockSpec((_XBLK, _PAD), lambda i: (i, 0)),
        ],
        out_specs=pl.BlockSpec((_XBLK, _ODIM), lambda i: (i, 0)),
        out_shape=jax.ShapeDtypeStruct((_BATCH, _ODIM), jnp.float32),
    )(x, pe16)
